# baseline (speedup 1.0000x reference)
import functools

import jax
import jax.numpy as jnp
from jax import lax
from jax.experimental import pallas as pl
from jax.experimental.pallas import tpu as pltpu
from jax.experimental.pallas import tpu_sc as plsc

NWORDS = 100000
EMB = 16
HID = 128
NHIST = 20
B = 1024

_TOTAL = B * NHIST
_NW = 32
_CHUNK = 128
_PER_W = _TOTAL // _NW
_NCH = _PER_W // _CHUNK
_BN = 1024


def _make_sc_gather():
    mesh = plsc.VectorSubcoreMesh(core_axis_name="c", subcore_axis_name="s")

    @functools.partial(
        pl.kernel,
        mesh=mesh,
        out_type=jax.ShapeDtypeStruct((_NW, _NCH, _CHUNK, EMB), jnp.float32),
        scratch_types=[
            pltpu.VMEM((_NCH, _CHUNK), jnp.int32),
            pltpu.VMEM((_NCH, _CHUNK, EMB), jnp.float32),
            pltpu.SemaphoreType.DMA,
        ],
        compiler_params=pltpu.CompilerParams(use_tc_tiling_on_sc=False),
    )
    def sc_gather(idx_hbm, table_hbm, out_hbm, idx_v, rows_v, sem):
        wid = lax.axis_index("s") * 2 + lax.axis_index("c")
        pltpu.sync_copy(idx_hbm.at[wid], idx_v)
        copies = [
            pltpu.async_copy(table_hbm.at[idx_v.at[j]], rows_v.at[j], sem)
            for j in range(_NCH)
        ]
        for c in copies:
            c.wait()
        pltpu.sync_copy(rows_v, out_hbm.at[wid])

    return sc_gather


_sc_gather_cache = []


def _sc_gather(idx, table):
    if not _sc_gather_cache:
        _sc_gather_cache.append(_make_sc_gather())
    return _sc_gather_cache[0](idx, table)


def _mlp_body(feat_ref, w1_ref, b1_ref, w2_ref, b2_ref, out_ref, h_ref):
    @pl.when(pl.program_id(0) == 0)
    def _():
        h = lax.dot_general(
            feat_ref[...], w1_ref[...],
            (((1,), (1,)), ((), ())),
            preferred_element_type=jnp.float32,
        )
        h_ref[...] = h + b1_ref[...]

    out_ref[...] = lax.dot_general(
        h_ref[...], w2_ref[...],
        (((1,), (1,)), ((), ())),
        preferred_element_type=jnp.float32,
    ) + b2_ref[...]


def _mlp(feat, W1, b1, W2, b2):
    nt = pl.cdiv(NWORDS, _BN)
    return pl.pallas_call(
        _mlp_body,
        grid=(nt,),
        in_specs=[
            pl.BlockSpec((B, NHIST * EMB), lambda i: (0, 0)),
            pl.BlockSpec((HID, NHIST * EMB), lambda i: (0, 0)),
            pl.BlockSpec((1, HID), lambda i: (0, 0)),
            pl.BlockSpec((_BN, HID), lambda i: (i, 0)),
            pl.BlockSpec((1, _BN), lambda i: (0, i)),
        ],
        out_specs=pl.BlockSpec((B, _BN), lambda i: (0, i)),
        out_shape=jax.ShapeDtypeStruct((B, NWORDS), jnp.float32),
        scratch_shapes=[pltpu.VMEM((B, HID), jnp.float32)],
    )(feat, W1, b1.reshape(1, HID), W2, b2.reshape(1, NWORDS))


def kernel(x, emb_table, W1, b1, W2, b2):
    idx = x.astype(jnp.int32).reshape(_NW, _NCH, _CHUNK)
    rows = _sc_gather(idx, emb_table)
    feat = rows.reshape(B, NHIST * EMB)
    return _mlp(feat, W1, b1, W2, b2)


# speedup vs baseline: 1.1586x; 1.1586x over previous
import functools

import jax
import jax.numpy as jnp
from jax import lax
from jax.experimental import pallas as pl
from jax.experimental.pallas import tpu as pltpu
from jax.experimental.pallas import tpu_sc as plsc

NWORDS = 100000
EMB = 16
HID = 128
NHIST = 20
B = 1024

_TOTAL = B * NHIST
_NW = 32
_CHUNK = 128
_PER_W = _TOTAL // _NW
_NCH = _PER_W // _CHUNK
_BN = 2048
_NT = -(-NWORDS // _BN)
_TAIL = NWORDS - (_NT - 1) * _BN
_NBUF = 4


def _make_sc_gather():
    mesh = plsc.VectorSubcoreMesh(core_axis_name="c", subcore_axis_name="s")

    @functools.partial(
        pl.kernel,
        mesh=mesh,
        out_type=jax.ShapeDtypeStruct((_NW, _NCH, _CHUNK, EMB), jnp.float32),
        scratch_types=[
            pltpu.VMEM((_NCH, _CHUNK), jnp.int32),
            pltpu.VMEM((_NCH, _CHUNK, EMB), jnp.float32),
            pltpu.SemaphoreType.DMA,
        ],
        compiler_params=pltpu.CompilerParams(use_tc_tiling_on_sc=False),
    )
    def sc_gather(idx_hbm, table_hbm, out_hbm, idx_v, rows_v, sem):
        wid = lax.axis_index("s") * 2 + lax.axis_index("c")
        pltpu.sync_copy(idx_hbm.at[wid], idx_v)
        copies = [
            pltpu.async_copy(table_hbm.at[idx_v.at[j]], rows_v.at[j], sem)
            for j in range(_NCH)
        ]
        for c in copies:
            c.wait()
        pltpu.sync_copy(rows_v, out_hbm.at[wid])

    return sc_gather


_sc_gather_cache = []


def _sc_gather(idx, table):
    if not _sc_gather_cache:
        _sc_gather_cache.append(_make_sc_gather())
    return _sc_gather_cache[0](idx, table)


def _mlp_body(feat_ref, w1_ref, b1_ref, w2_ref, b2_ref, out_ref, tail_ref,
              h_ref, acc_ref, sem):
    i = pl.program_id(0)

    @pl.when(i == 0)
    def _():
        h = lax.dot_general(
            feat_ref[...], w1_ref[...],
            (((1,), (1,)), ((), ())),
            preferred_element_type=jnp.float32,
        )
        h_ref[...] = h + b1_ref[...]

    slot = lax.rem(i, _NBUF)

    @pl.when(i >= _NBUF)
    def _():
        pltpu.make_async_copy(
            acc_ref.at[slot], out_ref.at[:, pl.ds(0, _BN)], sem.at[slot]
        ).wait()

    acc_ref[slot] = lax.dot_general(
        h_ref[...], w2_ref[...],
        (((1,), (1,)), ((), ())),
        preferred_element_type=jnp.float32,
    ) + b2_ref[...]

    @pl.when(i < _NT - 1)
    def _():
        pltpu.make_async_copy(
            acc_ref.at[slot], out_ref.at[:, pl.ds(i * _BN, _BN)], sem.at[slot]
        ).start()

    @pl.when(i == _NT - 1)
    def _():
        pltpu.make_async_copy(
            acc_ref.at[slot], tail_ref, sem.at[slot]
        ).start()
        for k in range(_NBUF - 1, 0, -1):
            s = (_NT - 1 - k) % _NBUF
            pltpu.make_async_copy(
                acc_ref.at[s], out_ref.at[:, pl.ds(0, _BN)], sem.at[s]
            ).wait()
        pltpu.make_async_copy(
            acc_ref.at[(_NT - 1) % _NBUF], tail_ref, sem.at[(_NT - 1) % _NBUF]
        ).wait()


def _mlp(feat, W1, b1, W2, b2):
    return pl.pallas_call(
        _mlp_body,
        grid=(_NT,),
        in_specs=[
            pl.BlockSpec((B, NHIST * EMB), lambda i: (0, 0)),
            pl.BlockSpec((HID, NHIST * EMB), lambda i: (0, 0)),
            pl.BlockSpec((1, HID), lambda i: (0, 0)),
            pl.BlockSpec((_BN, HID), lambda i: (i, 0)),
            pl.BlockSpec((1, _BN), lambda i: (0, i)),
        ],
        out_specs=[
            pl.BlockSpec(memory_space=pl.ANY),
            pl.BlockSpec(memory_space=pl.ANY),
        ],
        out_shape=[
            jax.ShapeDtypeStruct((B, NWORDS), jnp.float32),
            jax.ShapeDtypeStruct((B, _BN), jnp.float32),
        ],
        scratch_shapes=[
            pltpu.VMEM((B, HID), jnp.float32),
            pltpu.VMEM((_NBUF, B, _BN), jnp.float32),
            pltpu.SemaphoreType.DMA((_NBUF,)),
        ],
        compiler_params=pltpu.CompilerParams(
            vmem_limit_bytes=100 * 1024 * 1024,
        ),
    )(feat, W1, b1.reshape(1, HID), W2, b2.reshape(1, NWORDS))


def kernel(x, emb_table, W1, b1, W2, b2):
    idx = x.astype(jnp.int32).reshape(_NW, _NCH, _CHUNK)
    rows = _sc_gather(idx, emb_table)
    feat = rows.reshape(B, NHIST * EMB)
    out, tail = _mlp(feat, W1, b1, W2, b2)
    return lax.dynamic_update_slice(
        out, tail[:, :_TAIL], (0, (_NT - 1) * _BN)
    )


# speedup vs baseline: 2.0711x; 1.7876x over previous
import functools

import jax
import jax.numpy as jnp
from jax import lax
from jax.experimental import pallas as pl
from jax.experimental.pallas import tpu as pltpu
from jax.experimental.pallas import tpu_sc as plsc

NWORDS = 100000
EMB = 16
HID = 128
NHIST = 20
B = 1024

_NW = 32
_BPW = B // _NW
_BN = 2048
_NT = -(-NWORDS // _BN)


def _make_sc_gather():
    mesh = plsc.VectorSubcoreMesh(core_axis_name="c", subcore_axis_name="s")

    @functools.partial(
        pl.kernel,
        mesh=mesh,
        out_type=jax.ShapeDtypeStruct((B, NHIST, EMB), jnp.float32),
        scratch_types=[
            pltpu.VMEM((_BPW, NHIST), jnp.int32),
            pltpu.VMEM((_BPW, NHIST, EMB), jnp.float32),
            pltpu.SemaphoreType.DMA,
        ],
        compiler_params=pltpu.CompilerParams(use_tc_tiling_on_sc=False),
    )
    def sc_gather(idx_hbm, table_hbm, out_hbm, idx_v, rows_v, sem):
        wid = lax.axis_index("s") * 2 + lax.axis_index("c")
        base = wid * _BPW
        pltpu.sync_copy(idx_hbm.at[pl.ds(base, _BPW)], idx_v)
        copies = [
            pltpu.async_copy(table_hbm.at[idx_v.at[b]], rows_v.at[b], sem)
            for b in range(_BPW)
        ]
        for c in copies:
            c.wait()
        pltpu.sync_copy(rows_v, out_hbm.at[pl.ds(base, _BPW)])

    return sc_gather


_sc_gather_cache = []


def _sc_gather(idx, table):
    if not _sc_gather_cache:
        _sc_gather_cache.append(_make_sc_gather())
    return _sc_gather_cache[0](idx, table)


def _mlp_body(feat_ref, w1_ref, b1_ref, w2_ref, b2_ref, out_ref, h_ref):
    @pl.when(pl.program_id(0) == 0)
    def _():
        w1 = w1_ref[...]
        h = b1_ref[...]
        for t in range(NHIST):
            h = h + lax.dot_general(
                feat_ref[:, t, :],
                lax.slice(w1, (0, t * EMB), (HID, (t + 1) * EMB)),
                (((1,), (1,)), ((), ())),
                preferred_element_type=jnp.float32,
            )
        h_ref[...] = h

    out_ref[...] = lax.dot_general(
        w2_ref[...], h_ref[...],
        (((1,), (1,)), ((), ())),
        preferred_element_type=jnp.float32,
    ) + b2_ref[...]


def _mlp(feat3, W1, b1, W2, b2):
    outT = pl.pallas_call(
        _mlp_body,
        grid=(_NT,),
        in_specs=[
            pl.BlockSpec((B, NHIST, EMB), lambda i: (0, 0, 0)),
            pl.BlockSpec((HID, NHIST * EMB), lambda i: (0, 0)),
            pl.BlockSpec((1, HID), lambda i: (0, 0)),
            pl.BlockSpec((_BN, HID), lambda i: (i, 0)),
            pl.BlockSpec((_BN, 1), lambda i: (i, 0)),
        ],
        out_specs=pl.BlockSpec((_BN, B), lambda i: (i, 0)),
        out_shape=jax.ShapeDtypeStruct((NWORDS, B), jnp.float32),
        scratch_shapes=[pltpu.VMEM((B, HID), jnp.float32)],
        compiler_params=pltpu.CompilerParams(
            vmem_limit_bytes=100 * 1024 * 1024,
        ),
    )(feat3, W1, b1.reshape(1, HID), W2, b2.reshape(NWORDS, 1))
    return outT.T


def kernel(x, emb_table, W1, b1, W2, b2):
    feat3 = _sc_gather(x.astype(jnp.int32), emb_table)
    return _mlp(feat3, W1, b1, W2, b2)


# speedup vs baseline: 2.2198x; 1.0718x over previous
import functools

import jax
import jax.numpy as jnp
from jax import lax
from jax.experimental import pallas as pl
from jax.experimental.pallas import tpu as pltpu
from jax.experimental.pallas import tpu_sc as plsc

NWORDS = 100000
EMB = 16
HID = 128
NHIST = 20
B = 1024

_TOTAL = B * NHIST
_NW = 32
_CHUNK = 128
_PER_W = _TOTAL // _NW
_NCH = _PER_W // _CHUNK
_BN = 4096
_NT = -(-NWORDS // _BN)
_TAIL = NWORDS - (_NT - 1) * _BN
_NBUF = 4


def _make_sc_gather():
    mesh = plsc.VectorSubcoreMesh(core_axis_name="c", subcore_axis_name="s")

    @functools.partial(
        pl.kernel,
        mesh=mesh,
        out_type=jax.ShapeDtypeStruct((_NW, _NCH, _CHUNK, EMB), jnp.float32),
        scratch_types=[
            pltpu.VMEM((_NCH, _CHUNK), jnp.int32),
            pltpu.VMEM((_NCH, _CHUNK, EMB), jnp.float32),
            pltpu.SemaphoreType.DMA,
        ],
        compiler_params=pltpu.CompilerParams(use_tc_tiling_on_sc=False),
    )
    def sc_gather(idx_hbm, table_hbm, out_hbm, idx_v, rows_v, sem):
        wid = lax.axis_index("s") * 2 + lax.axis_index("c")
        pltpu.sync_copy(idx_hbm.at[wid], idx_v)
        copies = [
            pltpu.async_copy(table_hbm.at[idx_v.at[j]], rows_v.at[j], sem)
            for j in range(_NCH)
        ]
        for c in copies:
            c.wait()
        pltpu.sync_copy(rows_v, out_hbm.at[wid])

    return sc_gather


_sc_gather_cache = []


def _sc_gather(idx, table):
    if not _sc_gather_cache:
        _sc_gather_cache.append(_make_sc_gather())
    return _sc_gather_cache[0](idx, table)


def _mlp_body(feat_ref, w1_ref, b1_ref, w2_ref, b2_ref, out_ref, h_ref):
    @pl.when(pl.program_id(0) == 0)
    def _():
        h = lax.dot_general(
            feat_ref[...], w1_ref[...],
            (((1,), (1,)), ((), ())),
            preferred_element_type=jnp.float32,
        )
        h_ref[...] = h + b1_ref[...]

    out_ref[...] = lax.dot_general(
        w2_ref[...], h_ref[...],
        (((1,), (1,)), ((), ())),
        preferred_element_type=jnp.float32,
    ) + b2_ref[...]


def _mlp(feat, W1, b1, W2, b2):
    outT = pl.pallas_call(
        _mlp_body,
        grid=(_NT,),
        in_specs=[
            pl.BlockSpec((B, NHIST * EMB), lambda i: (0, 0)),
            pl.BlockSpec((HID, NHIST * EMB), lambda i: (0, 0)),
            pl.BlockSpec((1, HID), lambda i: (0, 0)),
            pl.BlockSpec((_BN, HID), lambda i: (i, 0)),
            pl.BlockSpec((_BN, 1), lambda i: (i, 0)),
        ],
        out_specs=pl.BlockSpec((_BN, B), lambda i: (i, 0)),
        out_shape=jax.ShapeDtypeStruct((NWORDS, B), jnp.float32),
        scratch_shapes=[pltpu.VMEM((B, HID), jnp.float32)],
        compiler_params=pltpu.CompilerParams(
            vmem_limit_bytes=100 * 1024 * 1024,
        ),
    )(feat, W1, b1.reshape(1, HID), W2, b2.reshape(NWORDS, 1))
    return outT.T


def kernel(x, emb_table, W1, b1, W2, b2):
    idx = x.astype(jnp.int32).reshape(_NW, _NCH, _CHUNK)
    rows = _sc_gather(idx, emb_table)
    feat = rows.reshape(B, NHIST * EMB)
    return _mlp(feat, W1, b1, W2, b2)


# speedup vs baseline: 2.2239x; 1.0019x over previous
import functools

import jax
import jax.numpy as jnp
from jax import lax
from jax.experimental import pallas as pl
from jax.experimental.pallas import tpu as pltpu
from jax.experimental.pallas import tpu_sc as plsc

NWORDS = 100000
EMB = 16
HID = 128
NHIST = 20
B = 1024

_TOTAL = B * NHIST
_NW = 32
_CHUNK = 128
_PER_W = _TOTAL // _NW
_NCH = _PER_W // _CHUNK
_BN = 5120
_NT = -(-NWORDS // _BN)
_TAIL = NWORDS - (_NT - 1) * _BN
_NBUF = 4


def _make_sc_gather():
    mesh = plsc.VectorSubcoreMesh(core_axis_name="c", subcore_axis_name="s")

    @functools.partial(
        pl.kernel,
        mesh=mesh,
        out_type=jax.ShapeDtypeStruct((_NW, _NCH, _CHUNK, EMB), jnp.float32),
        scratch_types=[
            pltpu.VMEM((_NCH, _CHUNK), jnp.int32),
            pltpu.VMEM((_NCH, _CHUNK, EMB), jnp.float32),
            pltpu.SemaphoreType.DMA,
        ],
        compiler_params=pltpu.CompilerParams(use_tc_tiling_on_sc=False),
    )
    def sc_gather(idx_hbm, table_hbm, out_hbm, idx_v, rows_v, sem):
        wid = lax.axis_index("s") * 2 + lax.axis_index("c")
        pltpu.sync_copy(idx_hbm.at[wid], idx_v)
        copies = [
            pltpu.async_copy(table_hbm.at[idx_v.at[j]], rows_v.at[j], sem)
            for j in range(_NCH)
        ]
        for c in copies:
            c.wait()
        pltpu.sync_copy(rows_v, out_hbm.at[wid])

    return sc_gather


_sc_gather_cache = []


def _sc_gather(idx, table):
    if not _sc_gather_cache:
        _sc_gather_cache.append(_make_sc_gather())
    return _sc_gather_cache[0](idx, table)


def _mlp_body(feat_ref, w1_ref, b1_ref, w2_ref, b2_ref, out_ref, h_ref):
    @pl.when(pl.program_id(0) == 0)
    def _():
        h = lax.dot_general(
            feat_ref[...], w1_ref[...],
            (((1,), (1,)), ((), ())),
            preferred_element_type=jnp.float32,
        )
        h_ref[...] = h + b1_ref[...]

    out_ref[...] = lax.dot_general(
        w2_ref[...], h_ref[...],
        (((1,), (1,)), ((), ())),
        preferred_element_type=jnp.float32,
    ) + b2_ref[...]


def _mlp(feat, W1, b1, W2, b2):
    outT = pl.pallas_call(
        _mlp_body,
        grid=(_NT,),
        in_specs=[
            pl.BlockSpec((B, NHIST * EMB), lambda i: (0, 0)),
            pl.BlockSpec((HID, NHIST * EMB), lambda i: (0, 0)),
            pl.BlockSpec((1, HID), lambda i: (0, 0)),
            pl.BlockSpec((_BN, HID), lambda i: (i, 0)),
            pl.BlockSpec((_BN, 1), lambda i: (i, 0)),
        ],
        out_specs=pl.BlockSpec((_BN, B), lambda i: (i, 0)),
        out_shape=jax.ShapeDtypeStruct((NWORDS, B), jnp.float32),
        scratch_shapes=[pltpu.VMEM((B, HID), jnp.float32)],
        compiler_params=pltpu.CompilerParams(
            vmem_limit_bytes=100 * 1024 * 1024,
        ),
    )(feat, W1, b1.reshape(1, HID), W2, b2.reshape(NWORDS, 1))
    return outT.T


def kernel(x, emb_table, W1, b1, W2, b2):
    idx = x.astype(jnp.int32).reshape(_NW, _NCH, _CHUNK)
    rows = _sc_gather(idx, emb_table)
    feat = rows.reshape(B, NHIST * EMB)
    return _mlp(feat, W1, b1, W2, b2)
